# Initial kernel scaffold; baseline (speedup 1.0000x reference)
import jax
import jax.numpy as jnp
from jax.experimental import pallas as pl


def kernel(coords, grid):
    raise NotImplementedError("write your pallas kernel here")


# speedup vs baseline: 1.0662x; 1.0662x over previous
import functools

import jax
import jax.numpy as jnp
from jax import lax
from jax.experimental import pallas as pl
from jax.experimental.pallas import tpu as pltpu
from jax.experimental.pallas import tpu_sc as plsc

SIZE = 256
THR = 0.01
NC, NS, L = 2, 16, 16
NW = NC * NS

C = 1024
GLEN = 512


def _sc_kernel(N):
    PER_W = N // NW
    NCHUNK = PER_W // C
    G = C // L

    mesh = plsc.VectorSubcoreMesh(
        core_axis_name="c", subcore_axis_name="s",
        num_cores=NC, num_subcores=NS)

    @functools.partial(
        pl.kernel, mesh=mesh,
        out_type=jax.ShapeDtypeStruct((N,), jnp.int32),
        scratch_types=[
            pltpu.VMEM((C,), jnp.float32),
            pltpu.VMEM((C,), jnp.float32),
            pltpu.VMEM((C,), jnp.float32),
            pltpu.VMEM((8 * C,), jnp.int32),
            pltpu.VMEM((8 * C,), jnp.float32),
            pltpu.VMEM((8 * C,), jnp.float32),
            pltpu.VMEM((C,), jnp.int32),
            pltpu.SemaphoreType.DMA,
        ],
    )
    def k(xs_hbm, ys_hbm, zs_hbm, grid_hbm, out_hbm, xs_v, ys_v, zs_v,
          idx_v, w_v, vals_v, out_v, sem):
        wid = lax.axis_index("s") * NC + lax.axis_index("c")
        wbase = wid * PER_W
        lanes = lax.iota(jnp.int32, L)
        ones = jnp.full((L,), 1, jnp.int32)
        zerof = jnp.zeros((L,), jnp.float32)

        def chunk_body(ci, carry):
            base = wbase + ci * C
            pltpu.sync_copy(xs_hbm.at[pl.ds(base, C)], xs_v)
            pltpu.sync_copy(ys_hbm.at[pl.ds(base, C)], ys_v)
            pltpu.sync_copy(zs_hbm.at[pl.ds(base, C)], zs_v)

            def group_body(g, carry2):
                off0 = g * L

                def axis_comp(comp_v):
                    p = comp_v[pl.ds(off0, L)]
                    t = ((p + 1.0) * 256.0 - 1.0) / 2.0
                    ti = t.astype(jnp.int32)
                    tf = ti.astype(jnp.float32)
                    i0 = ti - jnp.where(t < tf, ones, 0)
                    w1 = t - i0.astype(jnp.float32)
                    w0 = 1.0 - w1
                    i1 = i0 + 1
                    w0 = jnp.where(i0 >= 0, w0, zerof)
                    w1 = jnp.where(i1 <= SIZE - 1, w1, zerof)
                    i0 = jnp.maximum(i0, 0)
                    i1 = jnp.minimum(i1, SIZE - 1)
                    return i0, i1, w0, w1

                x0, x1, wx0, wx1 = axis_comp(xs_v)
                y0, y1, wy0, wy1 = axis_comp(ys_v)
                z0, z1, wz0, wz1 = axis_comp(zs_v)
                off = g * L
                kc = 0
                for zi, wz in ((z0, wz0), (z1, wz1)):
                    for yi, wy in ((y0, wy0), (y1, wy1)):
                        zy = zi * (SIZE * SIZE) + yi * SIZE
                        wzy = wz * wy
                        for xi, wx in ((x0, wx0), (x1, wx1)):
                            idx_v[pl.ds(kc * C + off, L)] = zy + xi
                            w_v[pl.ds(kc * C + off, L)] = wzy * wx
                            kc += 1
                return carry2

            lax.fori_loop(0, G, group_body, 0)

            copies = []
            for o in range(0, 8 * C, GLEN):
                copies.append(pltpu.async_copy(
                    grid_hbm.at[idx_v.at[pl.ds(o, GLEN)]],
                    vals_v.at[pl.ds(o, GLEN)], sem))
            for cp in copies:
                cp.wait()

            def out_body(g, carry2):
                off = g * L
                acc = None
                for kc in range(8):
                    c = (vals_v[pl.ds(kc * C + off, L)]
                         * w_v[pl.ds(kc * C + off, L)])
                    acc = c if acc is None else acc + c
                out_v[pl.ds(off, L)] = jnp.where(acc > THR, ones, 0)
                return carry2

            lax.fori_loop(0, G, out_body, 0)

            pltpu.sync_copy(out_v, out_hbm.at[pl.ds(base, C)])
            return carry

        lax.fori_loop(0, NCHUNK, chunk_body, 0)

    return k


def kernel(coords, grid):
    n = coords.shape[0]
    out_i32 = _sc_kernel(n)(coords[:, 0], coords[:, 1], coords[:, 2],
                            grid.reshape(-1))
    return out_i32.astype(jnp.bool_)


# speedup vs baseline: 1.3627x; 1.2781x over previous
import functools

import jax
import jax.numpy as jnp
from jax import lax
from jax.experimental import pallas as pl
from jax.experimental.pallas import tpu as pltpu
from jax.experimental.pallas import tpu_sc as plsc

SIZE = 256
THR = 0.01
NC, NS, L = 2, 16, 16
NW = NC * NS

C = 1024
GLEN = 512


def _sc_kernel(N):
    PER_W = N // NW
    NCHUNK = PER_W // C
    NH = NCHUNK // 2
    G = C // L

    mesh = plsc.VectorSubcoreMesh(
        core_axis_name="c", subcore_axis_name="s",
        num_cores=NC, num_subcores=NS)

    buf_set = [
        pltpu.VMEM((C,), jnp.float32),
        pltpu.VMEM((C,), jnp.float32),
        pltpu.VMEM((C,), jnp.float32),
        pltpu.VMEM((8 * C,), jnp.int32),
        pltpu.VMEM((8 * C,), jnp.float32),
        pltpu.VMEM((8 * C,), jnp.float32),
        pltpu.VMEM((C,), jnp.int32),
        pltpu.SemaphoreType.DMA,
        pltpu.SemaphoreType.DMA,
        pltpu.SemaphoreType.DMA,
    ]

    @functools.partial(
        pl.kernel, mesh=mesh,
        out_type=jax.ShapeDtypeStruct((N,), jnp.int32),
        scratch_types=buf_set + buf_set,
    )
    def k(xs_hbm, ys_hbm, zs_hbm, grid_hbm, out_hbm, *scratch):
        b0, b1 = scratch[:10], scratch[10:]
        wid = lax.axis_index("s") * NC + lax.axis_index("c")
        wbase = wid * PER_W
        ones = jnp.full((L,), 1, jnp.int32)
        zerof = jnp.zeros((L,), jnp.float32)

        def cbase(ci):
            return wbase + jnp.where(ci < NCHUNK, ci, 0) * C

        def start_coords(ci, b):
            base = cbase(ci)
            xs_v, ys_v, zs_v, semc = b[0], b[1], b[2], b[7]
            pltpu.async_copy(xs_hbm.at[pl.ds(base, C)], xs_v, semc)
            pltpu.async_copy(ys_hbm.at[pl.ds(base, C)], ys_v, semc)
            pltpu.async_copy(zs_hbm.at[pl.ds(base, C)], zs_v, semc)

        def wait_coords(b):
            xs_v, ys_v, zs_v, semc = b[0], b[1], b[2], b[7]
            for v in (xs_v, ys_v, zs_v):
                pltpu.make_async_copy(xs_hbm.at[pl.ds(0, C)], v, semc).wait()

        def compute_idx(b):
            xs_v, ys_v, zs_v, idx_v, w_v = b[0], b[1], b[2], b[3], b[4]

            def group_body(g, carry):
                off = g * L

                def axis_comp(comp_v):
                    p = comp_v[pl.ds(off, L)]
                    t = ((p + 1.0) * 256.0 - 1.0) / 2.0
                    ti = t.astype(jnp.int32)
                    tf = ti.astype(jnp.float32)
                    i0 = ti - jnp.where(t < tf, ones, 0)
                    w1 = t - i0.astype(jnp.float32)
                    w0 = 1.0 - w1
                    i1 = i0 + 1
                    w0 = jnp.where(i0 >= 0, w0, zerof)
                    w1 = jnp.where(i1 <= SIZE - 1, w1, zerof)
                    i0 = jnp.maximum(i0, 0)
                    i1 = jnp.minimum(i1, SIZE - 1)
                    return i0, i1, w0, w1

                x0, x1, wx0, wx1 = axis_comp(xs_v)
                y0, y1, wy0, wy1 = axis_comp(ys_v)
                z0, z1, wz0, wz1 = axis_comp(zs_v)
                kc = 0
                for zi, wz in ((z0, wz0), (z1, wz1)):
                    for yi, wy in ((y0, wy0), (y1, wy1)):
                        zy = zi * (SIZE * SIZE) + yi * SIZE
                        wzy = wz * wy
                        for xi, wx in ((x0, wx0), (x1, wx1)):
                            idx_v[pl.ds(kc * C + off, L)] = zy + xi
                            w_v[pl.ds(kc * C + off, L)] = wzy * wx
                            kc += 1
                return carry

            lax.fori_loop(0, G, group_body, 0)

        def fire_gathers(b):
            idx_v, vals_v, semg = b[3], b[5], b[8]
            for o in range(0, 8 * C, GLEN):
                pltpu.async_copy(
                    grid_hbm.at[idx_v.at[pl.ds(o, GLEN)]],
                    vals_v.at[pl.ds(o, GLEN)], semg)

        def wait_gathers(b):
            idx_v, vals_v, semg = b[3], b[5], b[8]
            for o in range(0, 8 * C, GLEN):
                pltpu.make_async_copy(
                    grid_hbm.at[idx_v.at[pl.ds(o, GLEN)]],
                    vals_v.at[pl.ds(o, GLEN)], semg).wait()

        def pass2_out(ci, b, drain_prev):
            w_v, vals_v, out_v, semo = b[4], b[5], b[6], b[9]

            @pl.when(drain_prev)
            def _():
                pltpu.make_async_copy(
                    out_v, out_hbm.at[pl.ds(wbase, C)], semo).wait()

            def out_body(g, carry):
                off = g * L
                acc = None
                for kc in range(8):
                    c = (vals_v[pl.ds(kc * C + off, L)]
                         * w_v[pl.ds(kc * C + off, L)])
                    acc = c if acc is None else acc + c
                out_v[pl.ds(off, L)] = jnp.where(acc > THR, ones, 0)
                return carry

            lax.fori_loop(0, G, out_body, 0)
            pltpu.async_copy(out_v, out_hbm.at[pl.ds(cbase(ci), C)], semo)

        start_coords(0, b0)
        wait_coords(b0)
        compute_idx(b0)
        fire_gathers(b0)
        start_coords(1, b1)

        def body(j, carry):
            ca = 2 * j + 1
            cb = 2 * j + 2
            wait_coords(b1)
            compute_idx(b1)
            wait_gathers(b0)
            fire_gathers(b1)
            start_coords(cb, b0)
            pass2_out(2 * j, b0, j > 0)

            wait_coords(b0)
            compute_idx(b0)
            wait_gathers(b1)
            fire_gathers(b0)
            start_coords(cb + 1, b1)
            pass2_out(ca, b1, j > 0)
            return carry

        lax.fori_loop(0, NH, body, 0)

        wait_gathers(b0)
        wait_coords(b1)
        for b in (b0, b1):
            pltpu.make_async_copy(
                b[6], out_hbm.at[pl.ds(wbase, C)], b[9]).wait()

    return k


def kernel(coords, grid):
    n = coords.shape[0]
    out_i32 = _sc_kernel(n)(coords[:, 0], coords[:, 1], coords[:, 2],
                            grid.reshape(-1))
    return out_i32.astype(jnp.bool_)
